# baseline (speedup 1.0000x reference)
import functools

import jax
import jax.numpy as jnp
import numpy as np
from jax import lax
from jax.experimental import pallas as pl
from jax.experimental.pallas import tpu as pltpu
from jax.experimental.pallas import tpu_sc as plsc

N = 10000
E = 160000
D_NODE = 16
D_EDGE = 4
D_OUT = 8
D_EMB = 16
H = 64
SILU_NORM = 1.6790

NC = 2
NS = 16

G_WORKERS = NC * NS
G_REAL = E // G_WORKERS
G_CHUNK = 128
G_NCHUNK = 40
G_PAD = G_NCHUNK * G_CHUNK

_gather_mesh = plsc.VectorSubcoreMesh(core_axis_name="c", subcore_axis_name="s")
_SC_PARAMS = pltpu.CompilerParams(use_tc_tiling_on_sc=False)


@functools.partial(
    pl.kernel,
    out_type=jax.ShapeDtypeStruct((E, D_NODE), jnp.float32),
    mesh=_gather_mesh,
    compiler_params=_SC_PARAMS,
    scratch_types=[
        pltpu.VMEM((G_PAD,), jnp.int32),
        pltpu.VMEM((G_PAD, D_NODE), jnp.float32),
        pltpu.VMEM_SHARED((N, D_NODE), jnp.float32),
        pltpu.SemaphoreType.DMA,
        pltpu.SemaphoreType.DMA,
        pltpu.SemaphoreType.DMA,
        pltpu.SemaphoreType.DMA,
    ],
)
def _gather_kernel(node_hbm, idx_hbm, out_hbm, idx_v, rows_v, node_sh,
                   sem1, sem2, osem1, osem2):
    sid = lax.axis_index("s")
    wid = sid * NC + lax.axis_index("c")

    @pl.when(sid == 0)
    def _stage():
        pltpu.sync_copy(node_hbm, node_sh)

    pltpu.sync_copy(idx_hbm.at[wid], idx_v)
    plsc.subcore_barrier()
    half = G_PAD // 2
    out_half = G_REAL // 2
    base = wid * G_REAL
    c1 = pltpu.async_copy(node_sh.at[idx_v.at[pl.ds(0, half)]],
                          rows_v.at[pl.ds(0, half)], sem1)
    c2 = pltpu.async_copy(node_sh.at[idx_v.at[pl.ds(half, half)]],
                          rows_v.at[pl.ds(half, half)], sem2)
    c1.wait()
    o1 = pltpu.async_copy(rows_v.at[pl.ds(0, out_half)],
                          out_hbm.at[pl.ds(base, out_half)], osem1)
    c2.wait()
    o2 = pltpu.async_copy(rows_v.at[pl.ds(out_half, G_REAL - out_half)],
                          out_hbm.at[pl.ds(base + out_half,
                                           G_REAL - out_half)], osem2)
    o1.wait()
    o2.wait()


B_EDGE = 4000


CTOT = D_NODE * D_EDGE * D_OUT

_RX128 = np.repeat(np.eye(D_NODE, dtype=np.float32), D_OUT, axis=1)
_SEL128 = np.tile(np.eye(D_OUT, dtype=np.float32), (128 // D_OUT, 1))


def _tc_body(demb_ref, attr_ref, xg_ref, w1_ref, w2_ref, rx_ref, sel_ref,
             out_ref):
    h = jnp.dot(demb_ref[...], w1_ref[...], preferred_element_type=jnp.float32)
    h = h * jax.nn.sigmoid(h) * SILU_NORM
    hb = h.astype(jnp.bfloat16)
    a = attr_ref[...]
    ha = jnp.concatenate([hb * a[:, j:j + 1] for j in range(D_EDGE)], axis=1)
    s = jnp.dot(ha, w2_ref[...], preferred_element_type=jnp.float32)
    xr = jnp.dot(xg_ref[...], rx_ref[...], preferred_element_type=jnp.float32)
    s = s * xr
    out_ref[...] = jnp.dot(s, sel_ref[...],
                           preferred_element_type=jnp.float32)


def _tc_compute(demb, attr, xg, w1s, w2s):
    grid = (E // B_EDGE,)
    return pl.pallas_call(
        _tc_body,
        grid=grid,
        in_specs=[
            pl.BlockSpec((B_EDGE, D_EMB), lambda e: (e, 0)),
            pl.BlockSpec((B_EDGE, D_EDGE), lambda e: (e, 0)),
            pl.BlockSpec((B_EDGE, D_NODE), lambda e: (e, 0)),
            pl.BlockSpec((D_EMB, H), lambda e: (0, 0)),
            pl.BlockSpec((D_EDGE * H, 128), lambda e: (0, 0)),
            pl.BlockSpec((D_NODE, 128), lambda e: (0, 0)),
            pl.BlockSpec((128, D_OUT), lambda e: (0, 0)),
        ],
        out_specs=pl.BlockSpec((B_EDGE, D_OUT), lambda e: (e, 0)),
        out_shape=jax.ShapeDtypeStruct((E, D_OUT), jnp.float32),
        compiler_params=pltpu.CompilerParams(
            dimension_semantics=("arbitrary",),
        ),
    )(demb, attr, xg, w1s, w2s, _RX128, _SEL128)


S_WORKERS = NS
S_REAL = E // S_WORKERS
S_CHUNK = 128
S_NCHUNK = 80
S_PAD = S_NCHUNK * S_CHUNK

_scatter_mesh = plsc.VectorSubcoreMesh(
    core_axis_name="c", subcore_axis_name="s", num_cores=1)


@functools.partial(
    pl.kernel,
    out_type=jax.ShapeDtypeStruct((N, D_OUT), jnp.float32),
    mesh=_scatter_mesh,
    compiler_params=_SC_PARAMS,
    scratch_types=[
        pltpu.VMEM((S_NCHUNK, S_CHUNK), jnp.int32),
        pltpu.VMEM((S_PAD, D_OUT), jnp.float32),
        pltpu.VMEM_SHARED((N, D_OUT), jnp.float32),
    ],
)
def _scatter_kernel(ef_hbm, idx_hbm, zeros_hbm, out_hbm, idx_v, ef_v, table):
    wid = lax.axis_index("s")
    pltpu.sync_copy(idx_hbm.at[wid], idx_v)
    pltpu.sync_copy(ef_hbm.at[pl.ds(wid * S_REAL, S_REAL)],
                    ef_v.at[pl.ds(0, S_REAL)])
    pltpu.sync_copy(zeros_hbm.at[pl.ds(0, S_PAD - S_REAL)],
                    ef_v.at[pl.ds(S_REAL, S_PAD - S_REAL)])

    @pl.when(wid == 0)
    def _init():
        pltpu.sync_copy(zeros_hbm, table)

    plsc.subcore_barrier()

    def body(j, carry):
        pltpu.sync_copy(ef_v.at[pl.ds(j * S_CHUNK, S_CHUNK)],
                        table.at[idx_v.at[j]], add=True)
        return carry

    lax.fori_loop(0, S_NCHUNK, body, 0)
    plsc.subcore_barrier()

    @pl.when(wid == 0)
    def _writeout():
        pltpu.sync_copy(table, out_hbm)


def _pad_indices(idx, workers, real, nchunk, chunk):
    idx = idx.astype(jnp.int32).reshape(workers, real)
    idx = jnp.pad(idx, ((0, 0), (0, nchunk * chunk - real)))
    return idx.reshape(workers, nchunk, chunk)


@jax.jit
def kernel(node_input, edge_src, edge_dst, edge_attr, dist_embedding, W1, W2):
    idx_dst = _pad_indices(edge_dst, G_WORKERS, G_REAL, G_NCHUNK,
                           G_CHUNK).reshape(G_WORKERS, G_PAD)
    xg = _gather_kernel(node_input, idx_dst)

    w1s = (W1 * (1.0 / jnp.sqrt(jnp.float32(D_EMB)))).astype(jnp.bfloat16)
    w2s = ((W2 * (1.0 / 256.0))
           .reshape(H, D_NODE, D_EDGE, D_OUT)
           .transpose(2, 0, 1, 3)
           .reshape(D_EDGE * H, D_NODE * D_OUT)
           .astype(jnp.bfloat16))
    ef = _tc_compute(dist_embedding.astype(jnp.bfloat16),
                     edge_attr.astype(jnp.bfloat16), xg, w1s, w2s)

    idx_src = _pad_indices(edge_src, S_WORKERS, S_REAL, S_NCHUNK, S_CHUNK)
    zeros = jnp.zeros((N, D_OUT), jnp.float32)
    return _scatter_kernel(ef, idx_src, zeros)


# speedup vs baseline: 1.0711x; 1.0711x over previous
import functools

import jax
import jax.numpy as jnp
import numpy as np
from jax import lax
from jax.experimental import pallas as pl
from jax.experimental.pallas import tpu as pltpu
from jax.experimental.pallas import tpu_sc as plsc

N = 10000
E = 160000
D_NODE = 16
D_EDGE = 4
D_OUT = 8
D_EMB = 16
H = 64
SILU_NORM = 1.6790

NC = 2
NS = 16

G_WORKERS = NC * NS
G_REAL = E // G_WORKERS
G_CHUNK = 128
G_NCHUNK = 40
G_PAD = G_NCHUNK * G_CHUNK

_gather_mesh = plsc.VectorSubcoreMesh(core_axis_name="c", subcore_axis_name="s")
_SC_PARAMS = pltpu.CompilerParams(use_tc_tiling_on_sc=False)


@functools.partial(
    pl.kernel,
    out_type=jax.ShapeDtypeStruct((E, D_NODE), jnp.float32),
    mesh=_gather_mesh,
    compiler_params=_SC_PARAMS,
    scratch_types=[
        pltpu.VMEM((G_PAD,), jnp.int32),
        pltpu.VMEM((G_PAD, D_NODE), jnp.float32),
        pltpu.VMEM_SHARED((N, D_NODE), jnp.float32),
        pltpu.SemaphoreType.DMA,
        pltpu.SemaphoreType.DMA,
        pltpu.SemaphoreType.DMA,
        pltpu.SemaphoreType.DMA,
    ],
)
def _gather_kernel(node_hbm, idx_hbm, out_hbm, idx_v, rows_v, node_sh,
                   sem1, sem2, osem1, osem2):
    sid = lax.axis_index("s")
    wid = sid * NC + lax.axis_index("c")

    @pl.when(sid == 0)
    def _stage():
        pltpu.sync_copy(node_hbm, node_sh)

    pltpu.sync_copy(idx_hbm.at[wid], idx_v)
    plsc.subcore_barrier()
    half = G_PAD // 2
    out_half = G_REAL // 2
    base = wid * G_REAL
    c1 = pltpu.async_copy(node_sh.at[idx_v.at[pl.ds(0, half)]],
                          rows_v.at[pl.ds(0, half)], sem1)
    c2 = pltpu.async_copy(node_sh.at[idx_v.at[pl.ds(half, half)]],
                          rows_v.at[pl.ds(half, half)], sem2)
    c1.wait()
    o1 = pltpu.async_copy(rows_v.at[pl.ds(0, out_half)],
                          out_hbm.at[pl.ds(base, out_half)], osem1)
    c2.wait()
    o2 = pltpu.async_copy(rows_v.at[pl.ds(out_half, G_REAL - out_half)],
                          out_hbm.at[pl.ds(base + out_half,
                                           G_REAL - out_half)], osem2)
    o1.wait()
    o2.wait()


B_EDGE = 4000


CTOT = D_NODE * D_EDGE * D_OUT

_RX128 = np.repeat(np.eye(D_NODE, dtype=np.float32), D_OUT, axis=1)
_SEL128 = np.tile(np.eye(D_OUT, dtype=np.float32), (128 // D_OUT, 1))


def _tc_body(demb_ref, attr_ref, xg_ref, w1_ref, w2_ref, rx_ref, sel_ref,
             out_ref):
    h = jnp.dot(demb_ref[...], w1_ref[...], preferred_element_type=jnp.float32)
    h = h * jax.nn.sigmoid(h) * SILU_NORM
    hb = h.astype(jnp.bfloat16)
    a = attr_ref[...]
    s = jnp.dot(hb * a[:, 0:1], w2_ref[0:H],
                preferred_element_type=jnp.float32)
    for j in range(1, D_EDGE):
        s = s + jnp.dot(hb * a[:, j:j + 1], w2_ref[H * j:H * (j + 1)],
                        preferred_element_type=jnp.float32)
    xr = jnp.dot(xg_ref[...], rx_ref[...], preferred_element_type=jnp.float32)
    s = s * xr
    out_ref[...] = jnp.dot(s, sel_ref[...],
                           preferred_element_type=jnp.float32)


def _tc_compute(demb, attr, xg, w1s, w2s):
    grid = (E // B_EDGE,)
    return pl.pallas_call(
        _tc_body,
        grid=grid,
        in_specs=[
            pl.BlockSpec((B_EDGE, D_EMB), lambda e: (e, 0)),
            pl.BlockSpec((B_EDGE, D_EDGE), lambda e: (e, 0)),
            pl.BlockSpec((B_EDGE, D_NODE), lambda e: (e, 0)),
            pl.BlockSpec((D_EMB, H), lambda e: (0, 0)),
            pl.BlockSpec((D_EDGE * H, 128), lambda e: (0, 0)),
            pl.BlockSpec((D_NODE, 128), lambda e: (0, 0)),
            pl.BlockSpec((128, D_OUT), lambda e: (0, 0)),
        ],
        out_specs=pl.BlockSpec((B_EDGE, D_OUT), lambda e: (e, 0)),
        out_shape=jax.ShapeDtypeStruct((E, D_OUT), jnp.float32),
        compiler_params=pltpu.CompilerParams(
            dimension_semantics=("arbitrary",),
        ),
    )(demb, attr, xg, w1s, w2s, _RX128, _SEL128)


S_WORKERS = NS
S_REAL = E // S_WORKERS
S_CHUNK = 128
S_NCHUNK = 80
S_PAD = S_NCHUNK * S_CHUNK

_scatter_mesh = plsc.VectorSubcoreMesh(
    core_axis_name="c", subcore_axis_name="s", num_cores=1)


@functools.partial(
    pl.kernel,
    out_type=jax.ShapeDtypeStruct((N, D_OUT), jnp.float32),
    mesh=_scatter_mesh,
    compiler_params=_SC_PARAMS,
    scratch_types=[
        pltpu.VMEM((S_NCHUNK, S_CHUNK), jnp.int32),
        pltpu.VMEM((S_PAD, D_OUT), jnp.float32),
        pltpu.VMEM_SHARED((N, D_OUT), jnp.float32),
    ],
)
def _scatter_kernel(ef_hbm, idx_hbm, zeros_hbm, out_hbm, idx_v, ef_v, table):
    wid = lax.axis_index("s")
    pltpu.sync_copy(idx_hbm.at[wid], idx_v)
    pltpu.sync_copy(ef_hbm.at[pl.ds(wid * S_REAL, S_REAL)],
                    ef_v.at[pl.ds(0, S_REAL)])
    pltpu.sync_copy(zeros_hbm.at[pl.ds(0, S_PAD - S_REAL)],
                    ef_v.at[pl.ds(S_REAL, S_PAD - S_REAL)])

    @pl.when(wid == 0)
    def _init():
        pltpu.sync_copy(zeros_hbm, table)

    plsc.subcore_barrier()

    def body(j, carry):
        pltpu.sync_copy(ef_v.at[pl.ds(j * S_CHUNK, S_CHUNK)],
                        table.at[idx_v.at[j]], add=True)
        return carry

    lax.fori_loop(0, S_NCHUNK, body, 0)
    plsc.subcore_barrier()

    @pl.when(wid == 0)
    def _writeout():
        pltpu.sync_copy(table, out_hbm)


def _pad_indices(idx, workers, real, nchunk, chunk):
    idx = idx.astype(jnp.int32).reshape(workers, real)
    idx = jnp.pad(idx, ((0, 0), (0, nchunk * chunk - real)))
    return idx.reshape(workers, nchunk, chunk)


@jax.jit
def kernel(node_input, edge_src, edge_dst, edge_attr, dist_embedding, W1, W2):
    idx_dst = _pad_indices(edge_dst, G_WORKERS, G_REAL, G_NCHUNK,
                           G_CHUNK).reshape(G_WORKERS, G_PAD)
    xg = _gather_kernel(node_input, idx_dst)

    w1s = (W1 * (1.0 / jnp.sqrt(jnp.float32(D_EMB)))).astype(jnp.bfloat16)
    w2s = ((W2 * (1.0 / 256.0))
           .reshape(H, D_NODE, D_EDGE, D_OUT)
           .transpose(2, 0, 1, 3)
           .reshape(D_EDGE * H, D_NODE * D_OUT)
           .astype(jnp.bfloat16))
    ef = _tc_compute(dist_embedding.astype(jnp.bfloat16),
                     edge_attr.astype(jnp.bfloat16), xg, w1s, w2s)

    idx_src = _pad_indices(edge_src, S_WORKERS, S_REAL, S_NCHUNK, S_CHUNK)
    zeros = jnp.zeros((N, D_OUT), jnp.float32)
    return _scatter_kernel(ef, idx_src, zeros)


# speedup vs baseline: 1.1203x; 1.0460x over previous
import functools

import jax
import jax.numpy as jnp
import numpy as np
from jax import lax
from jax.experimental import pallas as pl
from jax.experimental.pallas import tpu as pltpu
from jax.experimental.pallas import tpu_sc as plsc

N = 10000
E = 160000
D_NODE = 16
D_EDGE = 4
D_OUT = 8
D_EMB = 16
H = 64
SILU_NORM = 1.6790

NC = 2
NS = 16

G_WORKERS = NC * NS
G_REAL = E // G_WORKERS
G_CHUNK = 128
G_NCHUNK = 40
G_PAD = G_NCHUNK * G_CHUNK

_gather_mesh = plsc.VectorSubcoreMesh(core_axis_name="c", subcore_axis_name="s")
_SC_PARAMS = pltpu.CompilerParams(use_tc_tiling_on_sc=False)


@functools.partial(
    pl.kernel,
    out_type=jax.ShapeDtypeStruct((E, D_NODE), jnp.float32),
    mesh=_gather_mesh,
    compiler_params=_SC_PARAMS,
    scratch_types=[
        pltpu.VMEM((G_PAD,), jnp.int32),
        pltpu.VMEM((G_PAD, D_NODE), jnp.float32),
        pltpu.VMEM_SHARED((N, D_NODE), jnp.float32),
        pltpu.SemaphoreType.DMA,
        pltpu.SemaphoreType.DMA,
        pltpu.SemaphoreType.DMA,
        pltpu.SemaphoreType.DMA,
    ],
)
def _gather_kernel(node_hbm, idx_hbm, out_hbm, idx_v, rows_v, node_sh,
                   sem1, sem2, osem1, osem2):
    sid = lax.axis_index("s")
    wid = sid * NC + lax.axis_index("c")

    @pl.when(sid == 0)
    def _stage():
        pltpu.sync_copy(node_hbm, node_sh)

    pltpu.sync_copy(idx_hbm.at[wid], idx_v)
    plsc.subcore_barrier()
    half = G_PAD // 2
    out_half = G_REAL // 2
    base = wid * G_REAL
    c1 = pltpu.async_copy(node_sh.at[idx_v.at[pl.ds(0, half)]],
                          rows_v.at[pl.ds(0, half)], sem1)
    c2 = pltpu.async_copy(node_sh.at[idx_v.at[pl.ds(half, half)]],
                          rows_v.at[pl.ds(half, half)], sem2)
    c1.wait()
    o1 = pltpu.async_copy(rows_v.at[pl.ds(0, out_half)],
                          out_hbm.at[pl.ds(base, out_half)], osem1)
    c2.wait()
    o2 = pltpu.async_copy(rows_v.at[pl.ds(out_half, G_REAL - out_half)],
                          out_hbm.at[pl.ds(base + out_half,
                                           G_REAL - out_half)], osem2)
    o1.wait()
    o2.wait()


B_EDGE = 4000


CTOT = D_NODE * D_EDGE * D_OUT

_PERMJ = np.empty((CTOT,), np.int64)
for _c in range(CTOT):
    _j, _m = _c >> 7, _c & 127
    _i, _k = _m >> 3, _m & 7
    _PERMJ[_c] = _i * 32 + _j * 8 + _k
_RX128 = np.repeat(np.eye(D_NODE, dtype=np.float32), D_OUT, axis=1)
_SEL128 = np.tile(np.eye(D_OUT, dtype=np.float32), (128 // D_OUT, 1))


def _tc_body(demb_ref, attr_ref, xg_ref, w1_ref, w2_ref, rx_ref, sel_ref,
             out_ref):
    h = jnp.dot(demb_ref[...], w1_ref[...], preferred_element_type=jnp.float32)
    h = h * jax.nn.sigmoid(h) * SILU_NORM
    tpw = jnp.dot(h.astype(jnp.bfloat16), w2_ref[...],
                  preferred_element_type=jnp.float32)
    a = attr_ref[...]
    s = tpw[:, 0:128] * a[:, 0:1]
    for j in range(1, D_EDGE):
        s = s + tpw[:, 128 * j:128 * (j + 1)] * a[:, j:j + 1]
    xr = jnp.dot(xg_ref[...], rx_ref[...], preferred_element_type=jnp.float32)
    s = s * xr
    out_ref[...] = jnp.dot(s, sel_ref[...],
                           preferred_element_type=jnp.float32)


def _tc_compute(demb, attr, xg, w1s, w2s):
    grid = (E // B_EDGE,)
    return pl.pallas_call(
        _tc_body,
        grid=grid,
        in_specs=[
            pl.BlockSpec((B_EDGE, D_EMB), lambda e: (e, 0)),
            pl.BlockSpec((B_EDGE, D_EDGE), lambda e: (e, 0)),
            pl.BlockSpec((B_EDGE, D_NODE), lambda e: (e, 0)),
            pl.BlockSpec((D_EMB, H), lambda e: (0, 0)),
            pl.BlockSpec((H, CTOT), lambda e: (0, 0)),
            pl.BlockSpec((D_NODE, 128), lambda e: (0, 0)),
            pl.BlockSpec((128, D_OUT), lambda e: (0, 0)),
        ],
        out_specs=pl.BlockSpec((B_EDGE, D_OUT), lambda e: (e, 0)),
        out_shape=jax.ShapeDtypeStruct((E, D_OUT), jnp.float32),
        compiler_params=pltpu.CompilerParams(
            dimension_semantics=("arbitrary",),
        ),
    )(demb, attr, xg, w1s, w2s, _RX128, _SEL128)


S_WORKERS = NS
S_REAL = E // S_WORKERS
S_CHUNK = 128
S_NCHUNK = 80
S_PAD = S_NCHUNK * S_CHUNK

_scatter_mesh = plsc.VectorSubcoreMesh(
    core_axis_name="c", subcore_axis_name="s", num_cores=1)


@functools.partial(
    pl.kernel,
    out_type=jax.ShapeDtypeStruct((N, D_OUT), jnp.float32),
    mesh=_scatter_mesh,
    compiler_params=_SC_PARAMS,
    scratch_types=[
        pltpu.VMEM((S_NCHUNK, S_CHUNK), jnp.int32),
        pltpu.VMEM((S_PAD, D_OUT), jnp.float32),
        pltpu.VMEM_SHARED((N, D_OUT), jnp.float32),
    ],
)
def _scatter_kernel(ef_hbm, idx_hbm, zeros_hbm, out_hbm, idx_v, ef_v, table):
    wid = lax.axis_index("s")
    pltpu.sync_copy(idx_hbm.at[wid], idx_v)
    pltpu.sync_copy(ef_hbm.at[pl.ds(wid * S_REAL, S_REAL)],
                    ef_v.at[pl.ds(0, S_REAL)])
    pltpu.sync_copy(zeros_hbm.at[pl.ds(0, S_PAD - S_REAL)],
                    ef_v.at[pl.ds(S_REAL, S_PAD - S_REAL)])

    @pl.when(wid == 0)
    def _init():
        pltpu.sync_copy(zeros_hbm, table)

    plsc.subcore_barrier()

    def body(j, carry):
        pltpu.sync_copy(ef_v.at[pl.ds(j * S_CHUNK, S_CHUNK)],
                        table.at[idx_v.at[j]], add=True)
        return carry

    lax.fori_loop(0, S_NCHUNK, body, 0)
    plsc.subcore_barrier()

    @pl.when(wid == 0)
    def _writeout():
        pltpu.sync_copy(table, out_hbm)


def _pad_indices(idx, workers, real, nchunk, chunk):
    idx = idx.astype(jnp.int32).reshape(workers, real)
    idx = jnp.pad(idx, ((0, 0), (0, nchunk * chunk - real)))
    return idx.reshape(workers, nchunk, chunk)


@jax.jit
def kernel(node_input, edge_src, edge_dst, edge_attr, dist_embedding, W1, W2):
    idx_dst = _pad_indices(edge_dst, G_WORKERS, G_REAL, G_NCHUNK,
                           G_CHUNK).reshape(G_WORKERS, G_PAD)
    xg = _gather_kernel(node_input, idx_dst)

    w1s = (W1 * (1.0 / jnp.sqrt(jnp.float32(D_EMB)))).astype(jnp.bfloat16)
    w2s = (W2 * (1.0 / 256.0))[:, _PERMJ].astype(jnp.bfloat16)
    ef = _tc_compute(dist_embedding.astype(jnp.bfloat16),
                     edge_attr.astype(jnp.bfloat16), xg, w1s, w2s)

    idx_src = _pad_indices(edge_src, S_WORKERS, S_REAL, S_NCHUNK, S_CHUNK)
    zeros = jnp.zeros((N, D_OUT), jnp.float32)
    return _scatter_kernel(ef, idx_src, zeros)
